# baseline (speedup 1.0000x reference)
import functools

import jax
import jax.numpy as jnp
from jax import lax
from jax.experimental import pallas as pl
from jax.experimental.pallas import tpu as pltpu
from jax.experimental.pallas import tpu_sc as plsc

B, S, D, E = 4, 8192, 768, 8
NTOK = B * S
BLK = 4096
TBLK = 8192

NC, NS = 2, 16
NW = NC * NS
NH = 2
HTOK = NTOK // NH
TPW = HTOK // NW
CHUNK = 16


def _h_body(x_ref, wg_ref, bg_ref, wn_ref, bn_ref, eps_ref, h_ref):
    xb = x_ref[...]
    dn = (((1,), (1,)), ((), ()))
    gate = lax.dot_general(wg_ref[...], xb, dn,
                           preferred_element_type=jnp.float32) + bg_ref[...]
    noise = lax.dot_general(wn_ref[...], xb, dn,
                            preferred_element_type=jnp.float32) + bn_ref[...]
    h_ref[...] = gate + eps_ref[...].T * jax.nn.softplus(noise)


@functools.partial(
    pl.kernel,
    out_type=jax.ShapeDtypeStruct((E, HTOK), jnp.float32),
    mesh=plsc.VectorSubcoreMesh(core_axis_name="c", subcore_axis_name="s"),
    scratch_types=[
        pltpu.VMEM((E, TPW), jnp.float32),
        pltpu.VMEM((E, TPW), jnp.float32),
    ],
)
def _gate_sc(h_hbm, out_hbm, hbuf, gbuf):
    wid = lax.axis_index("s") * NC + lax.axis_index("c")
    base = wid * TPW
    pltpu.sync_copy(h_hbm.at[:, pl.ds(base, TPW)], hbuf)

    def chunk(c, carry):
        t0 = c * CHUNK
        v = [hbuf[e, pl.ds(t0, CHUNK)] for e in range(E)]
        m1 = v[0]
        for e in range(1, E):
            m1 = jnp.maximum(m1, v[e])
        i1 = jnp.full((CHUNK,), E, jnp.int32)
        for e in reversed(range(E)):
            i1 = jnp.where(v[e] == m1, jnp.int32(e), i1)
        h2 = [jnp.where(i1 == e, -jnp.inf, v[e]) for e in range(E)]
        m2 = h2[0]
        for e in range(1, E):
            m2 = jnp.maximum(m2, h2[e])
        i2 = jnp.full((CHUNK,), E, jnp.int32)
        for e in reversed(range(E)):
            i2 = jnp.where(h2[e] == m2, jnp.int32(e), i2)
        e2 = jnp.exp(m2 - m1)
        inv = 1.0 / (1.0 + e2)
        p2 = e2 * inv
        zero = jnp.zeros((CHUNK,), jnp.float32)
        for e in range(E):
            out_e = jnp.where(i1 == e, inv, jnp.where(i2 == e, p2, zero))
            gbuf[e, pl.ds(t0, CHUNK)] = out_e
        return carry

    lax.fori_loop(0, TPW // CHUNK, chunk, 0)
    pltpu.sync_copy(gbuf, out_hbm.at[:, pl.ds(base, TPW)])


def _t_body(g_ref, out_ref):
    out_ref[...] = g_ref[...].T


@jax.jit
def kernel(x, Wg, bg, Wnoise, bnoise, eps):
    x2 = x.reshape(NTOK, D)
    eps2 = eps.reshape(NTOK, E)
    halves = []
    for p in range(NH):
        off = p * (HTOK // BLK)
        h = pl.pallas_call(
            _h_body,
            grid=(HTOK // BLK,),
            in_specs=[
                pl.BlockSpec((BLK, D), lambda i, o=off: (i + o, 0)),
                pl.BlockSpec((E, D), lambda i: (0, 0)),
                pl.BlockSpec((E, 1), lambda i: (0, 0)),
                pl.BlockSpec((E, D), lambda i: (0, 0)),
                pl.BlockSpec((E, 1), lambda i: (0, 0)),
                pl.BlockSpec((BLK, E), lambda i, o=off: (i + o, 0)),
            ],
            out_specs=pl.BlockSpec((E, BLK), lambda i: (0, i)),
            out_shape=jax.ShapeDtypeStruct((E, HTOK), jnp.float32),
        )(x2, Wg, bg.reshape(E, 1), Wnoise, bnoise.reshape(E, 1), eps2)
        halves.append(_gate_sc(h))
    outs = [
        pl.pallas_call(
            _t_body,
            grid=(HTOK // TBLK,),
            in_specs=[pl.BlockSpec((E, TBLK), lambda i: (0, i))],
            out_specs=pl.BlockSpec((TBLK, E), lambda i: (i, 0)),
            out_shape=jax.ShapeDtypeStruct((HTOK, E), jnp.float32),
        )(gT)
        for gT in halves
    ]
    return jnp.concatenate(outs, axis=0).reshape(B, S, E)


# speedup vs baseline: 1.3237x; 1.3237x over previous
import functools

import jax
import jax.numpy as jnp
from jax import lax
from jax.experimental import pallas as pl
from jax.experimental.pallas import tpu as pltpu
from jax.experimental.pallas import tpu_sc as plsc

B, S, D, E = 4, 8192, 768, 8
NTOK = B * S
BLK = 4096
TBLK = 8192

NC, NS = 2, 16
NW = NC * NS
TPW = NTOK // NW
CHUNK = 16


def _h_body(x_ref, wg_ref, bg_ref, wn_ref, bn_ref, eps_ref, h_ref):
    xb = x_ref[...]
    dn = (((1,), (1,)), ((), ()))
    gate = lax.dot_general(wg_ref[...], xb, dn,
                           preferred_element_type=jnp.float32) + bg_ref[...]
    noise = lax.dot_general(wn_ref[...], xb, dn,
                            preferred_element_type=jnp.float32) + bn_ref[...]
    h_ref[...] = gate + eps_ref[...].T * jax.nn.softplus(noise)


@functools.partial(
    pl.kernel,
    out_type=jax.ShapeDtypeStruct((E, NTOK), jnp.float32),
    mesh=plsc.VectorSubcoreMesh(core_axis_name="c", subcore_axis_name="s"),
    scratch_types=[
        pltpu.VMEM((E, TPW), jnp.float32),
        pltpu.VMEM((E, TPW), jnp.float32),
    ],
)
def _gate_sc(h_hbm, out_hbm, hbuf, gbuf):
    wid = lax.axis_index("s") * NC + lax.axis_index("c")
    base = wid * TPW
    pltpu.sync_copy(h_hbm.at[:, pl.ds(base, TPW)], hbuf)

    def chunk(c, carry):
        t0 = c * CHUNK
        v = [hbuf[e, pl.ds(t0, CHUNK)] for e in range(E)]
        m1 = v[0]
        for e in range(1, E):
            m1 = jnp.maximum(m1, v[e])
        i1 = jnp.full((CHUNK,), E, jnp.int32)
        for e in reversed(range(E)):
            i1 = jnp.where(v[e] == m1, jnp.int32(e), i1)
        h2 = [jnp.where(i1 == e, -jnp.inf, v[e]) for e in range(E)]
        m2 = h2[0]
        for e in range(1, E):
            m2 = jnp.maximum(m2, h2[e])
        i2 = jnp.full((CHUNK,), E, jnp.int32)
        for e in reversed(range(E)):
            i2 = jnp.where(h2[e] == m2, jnp.int32(e), i2)
        e2 = jnp.exp(m2 - m1)
        inv = 1.0 / (1.0 + e2)
        p2 = e2 * inv
        zero = jnp.zeros((CHUNK,), jnp.float32)
        for e in range(E):
            out_e = jnp.where(i1 == e, inv, jnp.where(i2 == e, p2, zero))
            gbuf[e, pl.ds(t0, CHUNK)] = out_e
        return carry

    lax.fori_loop(0, TPW // CHUNK, chunk, 0)
    pltpu.sync_copy(gbuf, out_hbm.at[:, pl.ds(base, TPW)])


def _t_body(g_ref, out_ref):
    out_ref[...] = g_ref[...].T


@jax.jit
def kernel(x, Wg, bg, Wnoise, bnoise, eps):
    x2 = x.reshape(NTOK, D)
    eps2 = eps.reshape(NTOK, E)
    h = pl.pallas_call(
        _h_body,
        grid=(NTOK // BLK,),
        in_specs=[
            pl.BlockSpec((BLK, D), lambda i: (i, 0)),
            pl.BlockSpec((E, D), lambda i: (0, 0)),
            pl.BlockSpec((E, 1), lambda i: (0, 0)),
            pl.BlockSpec((E, D), lambda i: (0, 0)),
            pl.BlockSpec((E, 1), lambda i: (0, 0)),
            pl.BlockSpec((BLK, E), lambda i: (i, 0)),
        ],
        out_specs=pl.BlockSpec((E, BLK), lambda i: (0, i)),
        out_shape=jax.ShapeDtypeStruct((E, NTOK), jnp.float32),
    )(x2, Wg, bg.reshape(E, 1), Wnoise, bnoise.reshape(E, 1), eps2)
    gT = _gate_sc(h)
    return gT.T.reshape(B, S, E)
